# Initial kernel scaffold; baseline (speedup 1.0000x reference)
import jax
import jax.numpy as jnp
from jax.experimental import pallas as pl


def kernel(x, embeddings, w1, b1, w2, b2, w3, b3, embeddings_10, fc1_w, fc1_b, fc2_w, fc2_b, fc3_w, fc3_b, filter_mask_u):
    raise NotImplementedError("write your pallas kernel here")


# speedup vs baseline: 5.5234x; 5.5234x over previous
import numpy as np
import jax
import jax.numpy as jnp
from jax.experimental import pallas as pl


_B, _L, _N = 64, 30, 176
_E = 128
_T = _N * _L
_F = _T // 2 + 1
_J = 16
_P = _J * _N
_K = 8


def _consts():
    n = np.arange(_N)
    r = np.arange(_N)
    ang = 2.0 * np.pi * np.outer(n, r) / _N
    dft_c = np.cos(ang).astype(np.float32)
    dft_s = np.sin(ang).astype(np.float32)

    sT = 1.0 / np.sqrt(_T)
    f = np.arange(_P)
    l = np.arange(_L)
    beta = 2.0 * np.pi * np.outer(f, l) / _T
    tw_c = (np.cos(beta) * sT).astype(np.float32)
    tw_s = (np.sin(beta) * sT).astype(np.float32)

    wr = np.full((_P,), 2.0, dtype=np.float32)
    wi = np.full((_P,), 2.0, dtype=np.float32)
    wr[_F:] = 0.0
    wi[_F:] = 0.0
    wr[0] = 1.0
    wr[_T // 2] = 1.0
    wi[0] = 0.0
    wi[_T // 2] = 0.0

    idft_c = np.cos(ang).astype(np.float32)
    idft_s = np.sin(ang).astype(np.float32)
    return (dft_c, dft_s, tw_c, tw_s,
            wr.reshape(_P, 1), wi.reshape(_P, 1), idft_c, idft_s)


def _body(xt_ref, sv_ref, e10_ref, fc1_ref, fc1b_ref, fc2_ref, fc2b_ref,
          fc3_ref, fc3b_ref, dftc_ref, dfts_ref, twc_ref, tws_ref,
          wr_ref, wi_ref, idftc_ref, idfts_ref, out_ref):
    xb = xt_ref[0]

    xnr = jnp.dot(dftc_ref[...], xb, preferred_element_type=jnp.float32)
    xni = jnp.dot(dfts_ref[...], xb, preferred_element_type=jnp.float32)

    xnr_t = jnp.concatenate([xnr] * _J, axis=0)
    xni_t = jnp.concatenate([xni] * _J, axis=0)
    twc = twc_ref[...]
    tws = tws_ref[...]
    xcr = jnp.sum(xnr_t * twc - xni_t * tws, axis=1, keepdims=True)
    xci = -jnp.sum(xnr_t * tws + xni_t * twc, axis=1, keepdims=True)

    sv = sv_ref[...]
    emb = sv[0:1, :]
    u = sv[1:2, :]
    mask = ((u > 0.0) & (u < 1.0)).astype(jnp.float32)
    m = emb * mask
    g1r = m * sv[2:3, :]
    g1i = m * sv[3:4, :]
    d2r = sv[4:5, :]
    d2i = sv[5:6, :]
    d3r = sv[6:7, :]
    d3i = sv[7:8, :]
    b1r = sv[8:9, :]
    b1i = sv[9:10, :]
    b2r = sv[10:11, :]
    b2i = sv[11:12, :]
    b3r = sv[12:13, :]
    b3i = sv[13:14, :]

    o1r = jax.nn.relu(xcr * g1r - xci * g1i + b1r)
    o1i = jax.nn.relu(xci * g1r + xcr * g1i + b1i)
    zr = jax.nn.relu(o1r - 0.01)
    zi = jax.nn.relu(o1i - 0.01)
    o2r = jax.nn.relu(o1r * d2r - o1i * d2i + b2r)
    o2i = jax.nn.relu(o1i * d2r + o1r * d2i + b2i)
    zr = zr + jax.nn.relu(o2r - 0.01)
    zi = zi + jax.nn.relu(o2i - 0.01)
    o3r = jax.nn.relu(o2r * d3r - o2i * d3i + b3r)
    o3i = jax.nn.relu(o2i * d3r + o2r * d3i + b3i)
    zr = zr + jax.nn.relu(o3r - 0.01) + xcr * emb
    zi = zi + jax.nn.relu(o3i - 0.01) + xci * emb

    e10 = e10_ref[...]
    at = jnp.dot(twc, e10, preferred_element_type=jnp.float32)
    bt = jnp.dot(tws, e10, preferred_element_type=jnp.float32)
    wr = wr_ref[...]
    wi = wi_ref[...]
    a_wr = at * wr
    b_wi = bt * wi
    b_wr = bt * wr
    a_wi = at * wi

    uacc = jnp.zeros((_N, 64), dtype=jnp.float32)
    vacc = jnp.zeros((_N, 64), dtype=jnp.float32)
    for k in range(_K):
        pk = zr * a_wr[:, k:k + 1] - zi * b_wi[:, k:k + 1]
        qk = zr * b_wr[:, k:k + 1] + zi * a_wi[:, k:k + 1]
        uk = jnp.sum(pk.reshape(_J, _N, _E), axis=0)
        vk = jnp.sum(qk.reshape(_J, _N, _E), axis=0)
        w1k = fc1_ref[k * _E:(k + 1) * _E, :]
        uacc = uacc + jnp.dot(uk, w1k, preferred_element_type=jnp.float32)
        vacc = vacc + jnp.dot(vk, w1k, preferred_element_type=jnp.float32)

    h1 = (jnp.dot(idftc_ref[...], uacc, preferred_element_type=jnp.float32)
          - jnp.dot(idfts_ref[...], vacc, preferred_element_type=jnp.float32)
          + fc1b_ref[...])
    h1 = jnp.where(h1 >= 0, h1, 0.01 * h1)
    h2 = jnp.dot(h1, fc2_ref[...], preferred_element_type=jnp.float32) + fc2b_ref[...]
    h2 = jnp.where(h2 >= 0, h2, 0.01 * h2)

    v3 = jnp.mean(fc3_ref[...], axis=0, keepdims=True)
    c3 = jnp.mean(fc3b_ref[...])
    col = jnp.sum(h2 * v3, axis=1, keepdims=True) + c3
    out_ref[...] = col.reshape(1, _N, 1)


def kernel(x, embeddings, w1, b1, w2, b2, w3, b3, embeddings_10, fc1_w, fc1_b,
           fc2_w, fc2_b, fc3_w, fc3_b, filter_mask_u):
    dft_c, dft_s, tw_c, tw_s, wr, wi, idft_c, idft_s = _consts()

    xt = jnp.transpose(x, (0, 2, 1))
    sv = jnp.concatenate([
        embeddings,
        filter_mask_u.reshape(1, _E),
        jnp.diagonal(w1[0]).reshape(1, _E),
        jnp.diagonal(w1[1]).reshape(1, _E),
        jnp.diagonal(w2[0]).reshape(1, _E),
        jnp.diagonal(w2[1]).reshape(1, _E),
        jnp.diagonal(w3[0]).reshape(1, _E),
        jnp.diagonal(w3[1]).reshape(1, _E),
        b1[0].reshape(1, _E), b1[1].reshape(1, _E),
        b2[0].reshape(1, _E), b2[1].reshape(1, _E),
        b3[0].reshape(1, _E), b3[1].reshape(1, _E),
        jnp.zeros((2, _E), dtype=jnp.float32),
    ], axis=0)

    fc1_t = fc1_w.T.reshape(_E, _K, 64).transpose(1, 0, 2).reshape(_E * _K, 64)

    out_t = pl.pallas_call(
        _body,
        grid=(_B,),
        in_specs=[
            pl.BlockSpec((1, _N, _L), lambda b: (b, 0, 0)),
            pl.BlockSpec((16, _E), lambda b: (0, 0)),
            pl.BlockSpec((_L, _K), lambda b: (0, 0)),
            pl.BlockSpec((_E * _K, 64), lambda b: (0, 0)),
            pl.BlockSpec((1, 64), lambda b: (0, 0)),
            pl.BlockSpec((64, 256), lambda b: (0, 0)),
            pl.BlockSpec((1, 256), lambda b: (0, 0)),
            pl.BlockSpec((30, 256), lambda b: (0, 0)),
            pl.BlockSpec((1, 30), lambda b: (0, 0)),
            pl.BlockSpec((_N, _N), lambda b: (0, 0)),
            pl.BlockSpec((_N, _N), lambda b: (0, 0)),
            pl.BlockSpec((_P, _L), lambda b: (0, 0)),
            pl.BlockSpec((_P, _L), lambda b: (0, 0)),
            pl.BlockSpec((_P, 1), lambda b: (0, 0)),
            pl.BlockSpec((_P, 1), lambda b: (0, 0)),
            pl.BlockSpec((_N, _N), lambda b: (0, 0)),
            pl.BlockSpec((_N, _N), lambda b: (0, 0)),
        ],
        out_specs=pl.BlockSpec((1, _N, 1), lambda b: (b, 0, 0)),
        out_shape=jax.ShapeDtypeStruct((_B, _N, 1), jnp.float32),
    )(xt, sv, embeddings_10, fc1_t, fc1_b.reshape(1, 64),
      fc2_w.T, fc2_b.reshape(1, 256), fc3_w, fc3_b.reshape(1, 30),
      jnp.asarray(dft_c), jnp.asarray(dft_s), jnp.asarray(tw_c),
      jnp.asarray(tw_s), jnp.asarray(wr), jnp.asarray(wi),
      jnp.asarray(idft_c), jnp.asarray(idft_s))

    return out_t[:, :, 0]
